# baseline (speedup 1.0000x reference)
import functools

import jax
import jax.numpy as jnp
from jax import lax
from jax.experimental import pallas as pl
from jax.experimental.pallas import tpu as pltpu
from jax.experimental.pallas import tpu_sc as plsc

_B = 1024
_S = 50
_D = 128
_NTOK = _B * _S

_info = plsc.get_sparse_core_info()
_NC, _NS = _info.num_cores, _info.num_subcores
_NW = _NC * _NS
_PER_W = _NTOK // _NW
_C = 80
_NCHUNK = _PER_W // _C


def _gather_body(table_hbm, ids_hbm, out_hbm, idx_v, rows_v, gsem):
    wid = lax.axis_index("s") * _NC + lax.axis_index("c")
    pltpu.sync_copy(ids_hbm.at[wid], idx_v)
    base = wid * _PER_W
    for j in range(_NCHUNK):
        pltpu.async_copy(table_hbm.at[idx_v.at[j]], rows_v, gsem).wait()
        pltpu.sync_copy(rows_v, out_hbm.at[pl.ds(base + j * _C, _C)])


_gather = functools.partial(
    pl.kernel,
    mesh=plsc.VectorSubcoreMesh(core_axis_name="c", subcore_axis_name="s"),
    out_type=jax.ShapeDtypeStruct((_NTOK, _D), jnp.float32),
    scratch_types=[
        pltpu.VMEM((_NCHUNK, _C), jnp.int32),
        pltpu.VMEM((_C, _D), jnp.float32),
        pltpu.SemaphoreType.DMA,
    ],
)(_gather_body)


_MB = 64


def _mask_body(mask_ref, out_ref):
    m = mask_ref[...].astype(jnp.float32)
    out_ref[...] = jnp.broadcast_to(m[:, None, :], (_MB, _S, _S))


def _mask_kernel(mask):
    return pl.pallas_call(
        _mask_body,
        grid=(_B // _MB,),
        in_specs=[pl.BlockSpec((_MB, _S), lambda i: (i, 0))],
        out_specs=pl.BlockSpec((_MB, _S, _S), lambda i: (i, 0, 0)),
        out_shape=jax.ShapeDtypeStruct((_B, _S, _S), jnp.float32),
    )(mask)


def kernel(word_ids, mask, table):
    ids = word_ids.reshape(_NW, _NCHUNK, _C)
    emb = _gather(table, ids).reshape(_B, _S, _D)
    attention_mask = _mask_kernel(mask)
    return (emb, attention_mask)
